# Initial kernel scaffold; baseline (speedup 1.0000x reference)
import jax
import jax.numpy as jnp
from jax.experimental import pallas as pl


def kernel(input, cls_emb, rel_emb):
    raise NotImplementedError("write your pallas kernel here")


# speedup vs baseline: 2.0085x; 2.0085x over previous
import functools

import jax
import jax.numpy as jnp
from jax import lax
from jax.experimental import pallas as pl
from jax.experimental.pallas import tpu as pltpu
from jax.experimental.pallas import tpu_sc as plsc

EMB = 128
NCOL = 50
W = 128
B = 16384
NC = 2
NS = 16
NW = NC * NS
RPT = B // NW
CHUNK = 128
NCHUNK = RPT // CHUNK
GRP = 16
GPC = CHUNK // GRP
MARGIN = 0.1


def _newton_sqrt(s):
    i = plsc.bitcast(s, jnp.int32)
    y = plsc.bitcast((i >> 1) + jnp.int32(0x1FBD1DF5), jnp.float32)
    for _ in range(3):
        y = 0.5 * (y + s / y)
    return y


def _body(tab_hbm, i0_hbm, i2_hbm, out_hbm,
          i0_v, i2_v, cbuf, dbuf, outv, sem0, sem1):
    wid = lax.axis_index("s") * NC + lax.axis_index("c")
    sems = (sem0, sem1)

    pltpu.sync_copy(i0_hbm.at[wid], i0_v)
    pltpu.sync_copy(i2_hbm.at[wid], i2_v)

    def issue(k):
        slot = k % 2
        return (pltpu.async_copy(tab_hbm.at[i0_v.at[k]], cbuf.at[slot], sems[slot]),
                pltpu.async_copy(tab_hbm.at[i2_v.at[k]], dbuf.at[slot], sems[slot]))

    lane = lax.iota(jnp.int32, GRP)
    inflight = issue(0)

    for k in range(NCHUNK):
        slot = k % 2
        cur = inflight
        if k + 1 < NCHUNK:
            inflight = issue(k + 1)
        cur[0].wait()
        cur[1].wait()
        crows = cbuf.at[slot]
        drows = dbuf.at[slot]

        def group(g, carry):
            rid = lane + g * GRP
            s1 = jnp.zeros((GRP,), jnp.float32)
            s2 = jnp.zeros((GRP,), jnp.float32)
            s3 = jnp.zeros((GRP,), jnp.float32)
            for j in range(NCOL):
                cj = jnp.full((GRP,), j, jnp.int32)
                ck = jnp.full((GRP,), NCOL + j, jnp.int32)
                c1 = plsc.load_gather(crows, [rid, cj])
                d1 = plsc.load_gather(drows, [rid, cj])
                cr = jnp.abs(plsc.load_gather(crows, [rid, ck]))
                dr = jnp.abs(plsc.load_gather(drows, [rid, ck]))
                t1 = jnp.maximum(jnp.abs(c1 - d1) + cr - dr + MARGIN, 0.0)
                t2 = jnp.maximum(MARGIN - cr, 0.0)
                t3 = jnp.maximum(MARGIN - dr, 0.0)
                s1 = s1 + t1 * t1
                s2 = s2 + t2 * t2
                s3 = s3 + t3 * t3
            res = _newton_sqrt(s1) + _newton_sqrt(s2) + _newton_sqrt(s3)
            outv[pl.ds(k * CHUNK + g * GRP, GRP)] = res
            return carry

        lax.fori_loop(0, GPC, group, 0)

    pltpu.sync_copy(outv, out_hbm.at[wid])


@jax.jit
def _run(tab, i0, i2):
    mesh = plsc.VectorSubcoreMesh(
        core_axis_name="c", subcore_axis_name="s",
        num_cores=NC, num_subcores=NS)
    f = pl.kernel(
        _body,
        out_type=jax.ShapeDtypeStruct((NW, RPT), jnp.float32),
        mesh=mesh,
        compiler_params=pltpu.CompilerParams(needs_layout_passes=False),
        scratch_types=[
            pltpu.VMEM((NCHUNK, CHUNK), jnp.int32),
            pltpu.VMEM((NCHUNK, CHUNK), jnp.int32),
            pltpu.VMEM((2, CHUNK, W), jnp.float32),
            pltpu.VMEM((2, CHUNK, W), jnp.float32),
            pltpu.VMEM((RPT,), jnp.float32),
            pltpu.SemaphoreType.DMA,
            pltpu.SemaphoreType.DMA,
        ],
    )
    return f(tab, i0, i2)


def kernel(input, cls_emb, rel_emb):
    del rel_emb
    tab = jnp.concatenate(
        [cls_emb[:, :NCOL],
         cls_emb[:, EMB:EMB + NCOL],
         jnp.zeros((cls_emb.shape[0], W - 2 * NCOL), jnp.float32)], axis=1)
    i0 = input[:, 0].astype(jnp.int32).reshape(NW, NCHUNK, CHUNK)
    i2 = input[:, 2].astype(jnp.int32).reshape(NW, NCHUNK, CHUNK)
    out = _run(tab, i0, i2)
    return out.reshape(B, 1)
